# Initial kernel scaffold; baseline (speedup 1.0000x reference)
import functools

import jax
import jax.numpy as jnp
from jax import lax
from jax.experimental import pallas as pl
from jax.experimental.pallas import tpu as pltpu
from jax.experimental.pallas import tpu_sc as plsc

N_NODES = 10000
N_EDGES = 320000
D = 128
GRAPH_SIZE = 1000
NG = N_NODES // GRAPH_SIZE

NW = 32
NE_PER = N_EDGES // NW
VREGS = NE_PER // 16

BLK = 2560
NB = N_EDGES // BLK

_mesh = plsc.VectorSubcoreMesh(core_axis_name="c", subcore_axis_name="s")


@functools.partial(
    pl.kernel,
    mesh=_mesh,
    out_type=jax.ShapeDtypeStruct((4, N_EDGES), jnp.float32),
    scratch_types=[
        pltpu.VMEM((2 * N_NODES,), jnp.float32),
        pltpu.VMEM((NE_PER,), jnp.int32),
        pltpu.VMEM((NE_PER,), jnp.int32),
        pltpu.VMEM((NE_PER,), jnp.float32),
        pltpu.VMEM((NE_PER,), jnp.float32),
        pltpu.VMEM((NE_PER,), jnp.float32),
        pltpu.VMEM((NE_PER,), jnp.float32),
    ],
)
def _sc_gather(xflat_hbm, src_hbm, dst_hbm, u4_hbm,
               xflat_v, src_v, dst_v, u0_v, u1_v, u2_v, u3_v):
    wid = lax.axis_index("s") * 2 + lax.axis_index("c")
    base = wid * NE_PER
    pltpu.sync_copy(xflat_hbm, xflat_v)
    pltpu.sync_copy(src_hbm.at[pl.ds(base, NE_PER)], src_v)
    pltpu.sync_copy(dst_hbm.at[pl.ds(base, NE_PER)], dst_v)

    def body(i, _):
        sl = pl.ds(i * 16, 16)
        s2 = src_v[sl] * 2
        d2 = dst_v[sl] * 2
        u0_v[sl] = plsc.load_gather(xflat_v, [s2])
        u1_v[sl] = plsc.load_gather(xflat_v, [s2 + 1])
        u2_v[sl] = plsc.load_gather(xflat_v, [d2])
        u3_v[sl] = plsc.load_gather(xflat_v, [d2 + 1])
        return 0

    lax.fori_loop(0, VREGS, body, 0)
    pltpu.sync_copy(u0_v, u4_hbm.at[0, pl.ds(base, NE_PER)])
    pltpu.sync_copy(u1_v, u4_hbm.at[1, pl.ds(base, NE_PER)])
    pltpu.sync_copy(u2_v, u4_hbm.at[2, pl.ds(base, NE_PER)])
    pltpu.sync_copy(u3_v, u4_hbm.at[3, pl.ds(base, NE_PER)])


def _tc_alpha_body(u4_ref, ea_ref, v4t_ref, we_ref, att_ref, ex_ref):
    u = u4_ref[...]
    vt = v4t_ref[...]
    m = lax.dot_general(vt, u, (((1,), (0,)), ((), ())),
                        preferred_element_type=jnp.float32,
                        precision=lax.Precision.HIGHEST)
    m = m + we_ref[...] * ea_ref[...]
    m = jnp.where(m > 0, m, m * 0.2)
    t = m * att_ref[...]
    alpha = jnp.sum(t, axis=0, keepdims=True)
    ex_ref[...] = jnp.exp(alpha)


def _tc_alpha(u4, ea_rows, v4t, we_col, att_col):
    return pl.pallas_call(
        _tc_alpha_body,
        grid=(NB,),
        in_specs=[
            pl.BlockSpec((4, BLK), lambda i: (0, i)),
            pl.BlockSpec((1, BLK), lambda i: (i, 0)),
            pl.BlockSpec((D, 4), lambda i: (0, 0)),
            pl.BlockSpec((D, 1), lambda i: (0, 0)),
            pl.BlockSpec((D, 1), lambda i: (0, 0)),
        ],
        out_specs=pl.BlockSpec((1, BLK), lambda i: (i, 0)),
        out_shape=jax.ShapeDtypeStruct((NB, BLK), jnp.float32),
    )(u4, ea_rows, v4t, we_col, att_col)


@functools.partial(
    pl.kernel,
    mesh=_mesh,
    out_type=jax.ShapeDtypeStruct((3, NW, N_NODES), jnp.float32),
    scratch_types=[
        pltpu.VMEM((2 * N_NODES,), jnp.float32),
        pltpu.VMEM((NE_PER,), jnp.int32),
        pltpu.VMEM((NE_PER,), jnp.int32),
        pltpu.VMEM((NE_PER,), jnp.float32),
        pltpu.VMEM((3 * N_NODES,), jnp.float32),
    ],
)
def _sc_scatter(xflat_hbm, src_hbm, dst_hbm, ex_hbm, acc_hbm,
                xflat_v, src_v, dst_v, ex_v, acc_v):
    wid = lax.axis_index("s") * 2 + lax.axis_index("c")
    base = wid * NE_PER
    pltpu.sync_copy(xflat_hbm, xflat_v)
    pltpu.sync_copy(src_hbm.at[pl.ds(base, NE_PER)], src_v)
    pltpu.sync_copy(dst_hbm.at[pl.ds(base, NE_PER)], dst_v)
    pltpu.sync_copy(ex_hbm.at[pl.ds(base, NE_PER)], ex_v)

    def zero(i, _):
        acc_v[pl.ds(i * 16, 16)] = jnp.zeros((16,), jnp.float32)
        return 0

    lax.fori_loop(0, (3 * N_NODES) // 16, zero, 0)

    def body(i, _):
        sl = pl.ds(i * 16, 16)
        s2 = src_v[sl] * 2
        d = dst_v[sl]
        ex = ex_v[sl]
        xs0 = plsc.load_gather(xflat_v, [s2])
        xs1 = plsc.load_gather(xflat_v, [s2 + 1])
        plsc.addupdate_scatter(acc_v, [d], ex)
        plsc.addupdate_scatter(acc_v, [d + N_NODES], ex * xs0)
        plsc.addupdate_scatter(acc_v, [d + 2 * N_NODES], ex * xs1)
        return 0

    lax.fori_loop(0, VREGS, body, 0)
    pltpu.sync_copy(acc_v.at[pl.ds(0, N_NODES)], acc_hbm.at[0, wid])
    pltpu.sync_copy(acc_v.at[pl.ds(N_NODES, N_NODES)], acc_hbm.at[1, wid])
    pltpu.sync_copy(acc_v.at[pl.ds(2 * N_NODES, N_NODES)], acc_hbm.at[2, wid])


def _tc_bn_body(acc_ref, wl_ref, gamma_ref, beta_ref, zt_ref):
    a = acc_ref[...]
    r = jnp.sum(a, axis=1)
    denom = r[0:1, :]
    inv = 1.0 / (denom + 1e-16)
    sx = r[1:2, :] * inv
    sy = r[2:3, :] * inv
    mx = jnp.mean(sx, keepdims=True)
    my = jnp.mean(sy, keepdims=True)
    ux = sx - mx
    uy = sy - my
    c00 = jnp.mean(ux * ux, keepdims=True)
    c01 = jnp.mean(ux * uy, keepdims=True)
    c11 = jnp.mean(uy * uy, keepdims=True)
    w0 = wl_ref[0:1, :]
    w1 = wl_ref[1:2, :]
    var = c00 * w0 * w0 + 2.0 * c01 * w0 * w1 + c11 * w1 * w1
    scale = gamma_ref[...] / jnp.sqrt(var + 1e-5)
    wt = jnp.concatenate([w0 * scale, w1 * scale], axis=0)
    s_hat = jnp.concatenate([ux, uy], axis=0)
    zt = lax.dot_general(wt, s_hat, (((0,), (0,)), ((), ())),
                         preferred_element_type=jnp.float32,
                         precision=lax.Precision.HIGHEST)
    zt_ref[...] = jnp.maximum(zt + beta_ref[...], 0.0)


def _tc_bn(acc, w_l, gamma_row, beta_col):
    return pl.pallas_call(
        _tc_bn_body,
        in_specs=[
            pl.BlockSpec((3, NW, N_NODES), lambda: (0, 0, 0)),
            pl.BlockSpec((2, D), lambda: (0, 0)),
            pl.BlockSpec((1, D), lambda: (0, 0)),
            pl.BlockSpec((D, 1), lambda: (0, 0)),
        ],
        out_specs=pl.BlockSpec((D, N_NODES), lambda: (0, 0)),
        out_shape=jax.ShapeDtypeStruct((D, N_NODES), jnp.float32),
    )(acc, w_l, gamma_row, beta_col)


def _tc_decode_body(zt_ref, pi_ref):
    zg = zt_ref[...]
    lg = lax.dot_general(zg, zg, (((0,), (0,)), ((), ())),
                         preferred_element_type=jnp.float32,
                         precision=lax.Precision.HIGHEST)
    mx = jnp.max(lg, axis=0, keepdims=True)
    e = jnp.exp(lg - mx)
    s = jnp.sum(e, axis=0, keepdims=True)
    pi_ref[...] = (e / s)[None]


def _tc_decode(zt):
    return pl.pallas_call(
        _tc_decode_body,
        grid=(NG,),
        in_specs=[pl.BlockSpec((D, GRAPH_SIZE), lambda g: (0, g))],
        out_specs=pl.BlockSpec((1, GRAPH_SIZE, GRAPH_SIZE),
                               lambda g: (g, 0, 0)),
        out_shape=jax.ShapeDtypeStruct((NG, GRAPH_SIZE, GRAPH_SIZE),
                                       jnp.float32),
    )(zt)


def kernel(x, edge_index, edge_attributes, W_l, W_r, W_e, att, bias,
           bn_gamma, bn_beta):
    x = x.astype(jnp.float32)
    xflat = x.reshape(-1)
    src = edge_index[0].astype(jnp.int32)
    dst = edge_index[1].astype(jnp.int32)
    ea_rows = edge_attributes.astype(jnp.float32).reshape(NB, BLK)

    v4t = jnp.concatenate([W_l, W_r], axis=0).T
    we_col = W_e.reshape(1, D).T
    att_col = att.reshape(1, D).T

    u4 = _sc_gather(xflat, src, dst)
    ex = _tc_alpha(u4, ea_rows, v4t, we_col, att_col)
    acc = _sc_scatter(xflat, src, dst, ex.reshape(-1))
    zt = _tc_bn(acc, W_l.astype(jnp.float32),
                bn_gamma.reshape(1, D), bn_beta.reshape(D, 1))
    pi = _tc_decode(zt)
    return pi


# speedup vs baseline: 19.3053x; 19.3053x over previous
import functools

import jax
import jax.numpy as jnp
from jax import lax
from jax.experimental import pallas as pl
from jax.experimental.pallas import tpu as pltpu
from jax.experimental.pallas import tpu_sc as plsc

N_NODES = 10000
N_EDGES = 320000
D = 128
GRAPH_SIZE = 1000
NG = N_NODES // GRAPH_SIZE

NW = 32
NE_PER = N_EDGES // NW
VREGS = NE_PER // 16

BLK = 2560
NB = N_EDGES // BLK

_mesh = plsc.VectorSubcoreMesh(core_axis_name="c", subcore_axis_name="s")
_sc_params = pltpu.CompilerParams(needs_layout_passes=False)


@functools.partial(
    pl.kernel,
    mesh=_mesh,
    out_type=jax.ShapeDtypeStruct((4 * N_EDGES,), jnp.float32),
    compiler_params=_sc_params,
    scratch_types=[
        pltpu.VMEM((2 * N_NODES,), jnp.float32),
        pltpu.VMEM((NE_PER,), jnp.int32),
        pltpu.VMEM((NE_PER,), jnp.int32),
        pltpu.VMEM((NE_PER,), jnp.float32),
        pltpu.VMEM((NE_PER,), jnp.float32),
        pltpu.VMEM((NE_PER,), jnp.float32),
        pltpu.VMEM((NE_PER,), jnp.float32),
    ],
)
def _sc_gather(xflat_hbm, src_hbm, dst_hbm, u4_hbm,
               xflat_v, src_v, dst_v, u0_v, u1_v, u2_v, u3_v):
    wid = lax.axis_index("s") * 2 + lax.axis_index("c")
    base = wid * NE_PER
    pltpu.sync_copy(xflat_hbm, xflat_v)
    pltpu.sync_copy(src_hbm.at[pl.ds(base, NE_PER)], src_v)
    pltpu.sync_copy(dst_hbm.at[pl.ds(base, NE_PER)], dst_v)

    def body(i, _):
        sl = pl.ds(i * 16, 16)
        s2 = src_v[sl] * 2
        d2 = dst_v[sl] * 2
        u0_v[sl] = plsc.load_gather(xflat_v, [s2])
        u1_v[sl] = plsc.load_gather(xflat_v, [s2 + 1])
        u2_v[sl] = plsc.load_gather(xflat_v, [d2])
        u3_v[sl] = plsc.load_gather(xflat_v, [d2 + 1])
        return 0

    lax.fori_loop(0, VREGS, body, 0)
    pltpu.sync_copy(u0_v, u4_hbm.at[pl.ds(0 * N_EDGES + base, NE_PER)])
    pltpu.sync_copy(u1_v, u4_hbm.at[pl.ds(1 * N_EDGES + base, NE_PER)])
    pltpu.sync_copy(u2_v, u4_hbm.at[pl.ds(2 * N_EDGES + base, NE_PER)])
    pltpu.sync_copy(u3_v, u4_hbm.at[pl.ds(3 * N_EDGES + base, NE_PER)])


def _tc_alpha_body(u4_ref, ea_ref, v4t_ref, we_ref, att_ref, ex_ref):
    u = u4_ref[...]
    vt = v4t_ref[...]
    m = lax.dot_general(vt, u, (((1,), (0,)), ((), ())),
                        preferred_element_type=jnp.float32,
                        precision=lax.Precision.HIGHEST)
    m = m + we_ref[...] * ea_ref[0]
    m = jnp.where(m > 0, m, m * 0.2)
    t = m * att_ref[...]
    alpha = jnp.sum(t, axis=0, keepdims=True)
    ex_ref[0] = jnp.exp(alpha)


def _tc_alpha(u4, ea_rows, v4t, we_col, att_col):
    return pl.pallas_call(
        _tc_alpha_body,
        grid=(NB,),
        in_specs=[
            pl.BlockSpec((4, BLK), lambda i: (0, i)),
            pl.BlockSpec((1, 1, BLK), lambda i: (i, 0, 0)),
            pl.BlockSpec((D, 4), lambda i: (0, 0)),
            pl.BlockSpec((D, 1), lambda i: (0, 0)),
            pl.BlockSpec((D, 1), lambda i: (0, 0)),
        ],
        out_specs=pl.BlockSpec((1, 1, BLK), lambda i: (i, 0, 0)),
        out_shape=jax.ShapeDtypeStruct((NB, 1, BLK), jnp.float32),
    )(u4, ea_rows, v4t, we_col, att_col)


@functools.partial(
    pl.kernel,
    mesh=_mesh,
    out_type=jax.ShapeDtypeStruct((3 * NW * N_NODES,), jnp.float32),
    compiler_params=_sc_params,
    scratch_types=[
        pltpu.VMEM((2 * N_NODES,), jnp.float32),
        pltpu.VMEM((NE_PER,), jnp.int32),
        pltpu.VMEM((NE_PER,), jnp.int32),
        pltpu.VMEM((NE_PER,), jnp.float32),
        pltpu.VMEM((3 * N_NODES,), jnp.float32),
    ],
)
def _sc_scatter(xflat_hbm, src_hbm, dst_hbm, ex_hbm, acc_hbm,
                xflat_v, src_v, dst_v, ex_v, acc_v):
    wid = lax.axis_index("s") * 2 + lax.axis_index("c")
    base = wid * NE_PER
    pltpu.sync_copy(xflat_hbm, xflat_v)
    pltpu.sync_copy(src_hbm.at[pl.ds(base, NE_PER)], src_v)
    pltpu.sync_copy(dst_hbm.at[pl.ds(base, NE_PER)], dst_v)
    pltpu.sync_copy(ex_hbm.at[pl.ds(base, NE_PER)], ex_v)

    def zero(i, _):
        acc_v[pl.ds(i * 16, 16)] = jnp.zeros((16,), jnp.float32)
        return 0

    lax.fori_loop(0, (3 * N_NODES) // 16, zero, 0)

    def body(i, _):
        sl = pl.ds(i * 16, 16)
        s2 = src_v[sl] * 2
        d = dst_v[sl]
        ex = ex_v[sl]
        xs0 = plsc.load_gather(xflat_v, [s2])
        xs1 = plsc.load_gather(xflat_v, [s2 + 1])
        v1 = ex * xs0
        v2 = ex * xs1
        cnt, _ = plsc.scan_count(d)
        base = jnp.min(cnt)
        first = cnt == base
        plsc.addupdate_scatter(acc_v, [d], ex, mask=first)
        plsc.addupdate_scatter(acc_v, [d + N_NODES], v1, mask=first)
        plsc.addupdate_scatter(acc_v, [d + 2 * N_NODES], v2, mask=first)

        @pl.when(jnp.any(cnt != base))
        def _slow():
            def dup(j, _):
                m = cnt == base + j
                plsc.addupdate_scatter(acc_v, [d], ex, mask=m)
                plsc.addupdate_scatter(acc_v, [d + N_NODES], v1, mask=m)
                plsc.addupdate_scatter(acc_v, [d + 2 * N_NODES], v2, mask=m)
                return 0
            lax.fori_loop(1, 16, dup, 0)

        return 0

    lax.fori_loop(0, VREGS, body, 0)
    for c in range(3):
        pltpu.sync_copy(
            acc_v.at[pl.ds(c * N_NODES, N_NODES)],
            acc_hbm.at[pl.ds((c * NW + wid) * N_NODES, N_NODES)])


def _tc_bn_body(acc_ref, wl_ref, bias_ref, gamma_ref, beta_ref, zt_ref):
    a = acc_ref[...]
    r = jnp.sum(a, axis=1)
    denom = r[0:1, :]
    inv = 1.0 / (denom + 1e-16)
    sx = r[1:2, :] * inv
    sy = r[2:3, :] * inv
    s_t = jnp.concatenate([sx, sy], axis=0)
    out = lax.dot_general(wl_ref[...], s_t, (((0,), (0,)), ((), ())),
                          preferred_element_type=jnp.float32,
                          precision=lax.Precision.HIGHEST)
    out = out + bias_ref[...]
    mu = jnp.mean(out, axis=1, keepdims=True)
    ctr = out - mu
    var = jnp.mean(ctr * ctr, axis=1, keepdims=True)
    zt = ctr / jnp.sqrt(var + 1e-5) * gamma_ref[...] + beta_ref[...]
    zt_ref[...] = jnp.maximum(zt, 0.0)


def _tc_bn(acc, w_l, bias_col, gamma_col, beta_col):
    return pl.pallas_call(
        _tc_bn_body,
        in_specs=[
            pl.BlockSpec((3, NW, N_NODES), lambda: (0, 0, 0)),
            pl.BlockSpec((2, D), lambda: (0, 0)),
            pl.BlockSpec((D, 1), lambda: (0, 0)),
            pl.BlockSpec((D, 1), lambda: (0, 0)),
            pl.BlockSpec((D, 1), lambda: (0, 0)),
        ],
        out_specs=pl.BlockSpec((D, N_NODES), lambda: (0, 0)),
        out_shape=jax.ShapeDtypeStruct((D, N_NODES), jnp.float32),
    )(acc, w_l, bias_col, gamma_col, beta_col)


def _tc_decode_body(zt_ref, pi_ref):
    zg = zt_ref[...].reshape(D, GRAPH_SIZE).astype(jnp.bfloat16)
    lg = lax.dot_general(zg, zg, (((0,), (0,)), ((), ())),
                         preferred_element_type=jnp.float32)
    mx = jnp.max(lg, axis=0, keepdims=True)
    e = jnp.exp(lg - mx)
    s = jnp.sum(e, axis=0, keepdims=True)
    pi_ref[...] = (e / s)[None]


def _tc_decode(zt):
    return pl.pallas_call(
        _tc_decode_body,
        grid=(NG,),
        in_specs=[pl.BlockSpec((D, 1, 1, GRAPH_SIZE), lambda g: (0, g, 0, 0))],
        out_specs=pl.BlockSpec((1, GRAPH_SIZE, GRAPH_SIZE),
                               lambda g: (g, 0, 0)),
        out_shape=jax.ShapeDtypeStruct((NG, GRAPH_SIZE, GRAPH_SIZE),
                                       jnp.float32),
    )(zt)


def kernel(x, edge_index, edge_attributes, W_l, W_r, W_e, att, bias,
           bn_gamma, bn_beta):
    bf = lambda a: lax.optimization_barrier(
        a.astype(jnp.bfloat16)).astype(jnp.float32)
    xflat = bf(x).reshape(-1)
    src = edge_index[0].astype(jnp.int32)
    dst = edge_index[1].astype(jnp.int32)
    ea_rows = edge_attributes.astype(jnp.float32).reshape(NB, 1, BLK)
    wl_q = bf(W_l)

    v4t = jnp.concatenate([wl_q, bf(W_r)], axis=0).T
    we_col = W_e.astype(jnp.float32).reshape(1, D).T
    att_col = att.astype(jnp.float32).reshape(1, D).T

    u4 = _sc_gather(xflat, src, dst).reshape(4, N_EDGES)
    ex = _tc_alpha(u4, ea_rows, v4t, we_col, att_col)
    acc = _sc_scatter(xflat, src, dst,
                      ex.reshape(-1)).reshape(3, NW, N_NODES)
    zt = _tc_bn(acc, wl_q, bias.astype(jnp.float32).reshape(D, 1),
                bn_gamma.reshape(D, 1), bn_beta.reshape(D, 1))
    pi = _tc_decode(zt.reshape(D, NG, 1, GRAPH_SIZE))
    return pi


# speedup vs baseline: 25.0564x; 1.2979x over previous
import functools

import jax
import jax.numpy as jnp
from jax import lax
from jax.experimental import pallas as pl
from jax.experimental.pallas import tpu as pltpu
from jax.experimental.pallas import tpu_sc as plsc

N_NODES = 10000
N_EDGES = 320000
D = 128
GRAPH_SIZE = 1000
NG = N_NODES // GRAPH_SIZE

NW = 32
NE_PER = N_EDGES // NW
VREGS = NE_PER // 16

BLK = 2560
NB = N_EDGES // BLK

_mesh = plsc.VectorSubcoreMesh(core_axis_name="c", subcore_axis_name="s")
_sc_params = pltpu.CompilerParams(needs_layout_passes=False)


@functools.partial(
    pl.kernel,
    mesh=_mesh,
    out_type=jax.ShapeDtypeStruct((4 * N_EDGES,), jnp.float32),
    compiler_params=_sc_params,
    scratch_types=[
        pltpu.VMEM((2 * N_NODES,), jnp.float32),
        pltpu.VMEM((NE_PER,), jnp.int32),
        pltpu.VMEM((NE_PER,), jnp.int32),
        pltpu.VMEM((NE_PER,), jnp.float32),
        pltpu.VMEM((NE_PER,), jnp.float32),
        pltpu.VMEM((NE_PER,), jnp.float32),
        pltpu.VMEM((NE_PER,), jnp.float32),
    ],
)
def _sc_gather(xflat_hbm, src_hbm, dst_hbm, u4_hbm,
               xflat_v, src_v, dst_v, u0_v, u1_v, u2_v, u3_v):
    wid = lax.axis_index("s") * 2 + lax.axis_index("c")
    base = wid * NE_PER
    pltpu.sync_copy(xflat_hbm, xflat_v)
    pltpu.sync_copy(src_hbm.at[pl.ds(base, NE_PER)], src_v)
    pltpu.sync_copy(dst_hbm.at[pl.ds(base, NE_PER)], dst_v)

    def body(i, _):
        for j in range(5):
            sl = pl.ds((i * 5 + j) * 16, 16)
            s2 = src_v[sl] * 2
            d2 = dst_v[sl] * 2
            u0_v[sl] = plsc.load_gather(xflat_v, [s2])
            u1_v[sl] = plsc.load_gather(xflat_v, [s2 + 1])
            u2_v[sl] = plsc.load_gather(xflat_v, [d2])
            u3_v[sl] = plsc.load_gather(xflat_v, [d2 + 1])
        return 0

    lax.fori_loop(0, VREGS // 5, body, 0)
    pltpu.sync_copy(u0_v, u4_hbm.at[pl.ds(0 * N_EDGES + base, NE_PER)])
    pltpu.sync_copy(u1_v, u4_hbm.at[pl.ds(1 * N_EDGES + base, NE_PER)])
    pltpu.sync_copy(u2_v, u4_hbm.at[pl.ds(2 * N_EDGES + base, NE_PER)])
    pltpu.sync_copy(u3_v, u4_hbm.at[pl.ds(3 * N_EDGES + base, NE_PER)])


def _tc_alpha_body(u4_ref, ea_ref, v4t_ref, we_ref, att_ref, ex_ref):
    u = u4_ref[...]
    vt = v4t_ref[...]
    m = lax.dot_general(vt, u, (((1,), (0,)), ((), ())),
                        preferred_element_type=jnp.float32)
    m = m + we_ref[...] * ea_ref[0]
    m = jnp.where(m > 0, m, m * 0.2)
    t = m * att_ref[...]
    alpha = jnp.sum(t, axis=0, keepdims=True)
    ex_ref[0] = jnp.exp(alpha)


def _tc_alpha(u4, ea_rows, v4t, we_col, att_col):
    return pl.pallas_call(
        _tc_alpha_body,
        grid=(NB,),
        in_specs=[
            pl.BlockSpec((4, BLK), lambda i: (0, i)),
            pl.BlockSpec((1, 1, BLK), lambda i: (i, 0, 0)),
            pl.BlockSpec((D, 4), lambda i: (0, 0)),
            pl.BlockSpec((D, 1), lambda i: (0, 0)),
            pl.BlockSpec((D, 1), lambda i: (0, 0)),
        ],
        out_specs=pl.BlockSpec((1, 1, BLK), lambda i: (i, 0, 0)),
        out_shape=jax.ShapeDtypeStruct((NB, 1, BLK), jnp.float32),
    )(u4, ea_rows, v4t, we_col, att_col)


@functools.partial(
    pl.kernel,
    mesh=_mesh,
    out_type=jax.ShapeDtypeStruct((3 * NW * N_NODES,), jnp.float32),
    compiler_params=_sc_params,
    scratch_types=[
        pltpu.VMEM((2 * N_NODES,), jnp.float32),
        pltpu.VMEM((NE_PER,), jnp.int32),
        pltpu.VMEM((NE_PER,), jnp.int32),
        pltpu.VMEM((NE_PER,), jnp.float32),
        pltpu.VMEM((3 * N_NODES,), jnp.float32),
    ],
)
def _sc_scatter(xflat_hbm, src_hbm, dst_hbm, ex_hbm, acc_hbm,
                xflat_v, src_v, dst_v, ex_v, acc_v):
    wid = lax.axis_index("s") * 2 + lax.axis_index("c")
    base = wid * NE_PER
    pltpu.sync_copy(xflat_hbm, xflat_v)
    pltpu.sync_copy(src_hbm.at[pl.ds(base, NE_PER)], src_v)
    pltpu.sync_copy(dst_hbm.at[pl.ds(base, NE_PER)], dst_v)
    pltpu.sync_copy(ex_hbm.at[pl.ds(base, NE_PER)], ex_v)

    def zero(i, _):
        for j in range(15):
            acc_v[pl.ds((i * 15 + j) * 16, 16)] = jnp.zeros((16,), jnp.float32)
        return 0

    lax.fori_loop(0, (3 * N_NODES) // (16 * 15), zero, 0)

    def body(i, _):
        for j in range(5):
            sl = pl.ds((i * 5 + j) * 16, 16)
            s2 = src_v[sl] * 2
            d = dst_v[sl]
            ex = ex_v[sl]
            xs0 = plsc.load_gather(xflat_v, [s2])
            xs1 = plsc.load_gather(xflat_v, [s2 + 1])
            v1 = ex * xs0
            v2 = ex * xs1
            cnt, _ = plsc.scan_count(d)
            base = jnp.min(cnt)
            first = cnt == base
            plsc.addupdate_scatter(acc_v, [d], ex, mask=first)
            plsc.addupdate_scatter(acc_v, [d + N_NODES], v1, mask=first)
            plsc.addupdate_scatter(acc_v, [d + 2 * N_NODES], v2, mask=first)

            @pl.when(jnp.any(cnt != base))
            def _slow():
                def dup(k, _):
                    m = cnt == base + k
                    plsc.addupdate_scatter(acc_v, [d], ex, mask=m)
                    plsc.addupdate_scatter(acc_v, [d + N_NODES], v1, mask=m)
                    plsc.addupdate_scatter(acc_v, [d + 2 * N_NODES], v2,
                                           mask=m)
                    return 0
                lax.fori_loop(1, 16, dup, 0)
        return 0

    lax.fori_loop(0, VREGS // 5, body, 0)
    for c in range(3):
        pltpu.sync_copy(
            acc_v.at[pl.ds(c * N_NODES, N_NODES)],
            acc_hbm.at[pl.ds((c * NW + wid) * N_NODES, N_NODES)])


def _tc_bn_body(acc_ref, wl_ref, bias_ref, gamma_ref, beta_ref, zt_ref):
    a = acc_ref[...]
    r = jnp.sum(a, axis=1)
    denom = r[0:1, :]
    inv = 1.0 / (denom + 1e-16)
    sx = r[1:2, :] * inv
    sy = r[2:3, :] * inv
    s_t = jnp.concatenate([sx, sy], axis=0)
    out = lax.dot_general(wl_ref[...], s_t, (((0,), (0,)), ((), ())),
                          preferred_element_type=jnp.float32,
                          precision=lax.Precision.HIGHEST)
    out = out + bias_ref[...]
    mu = jnp.mean(out, axis=1, keepdims=True)
    ctr = out - mu
    var = jnp.mean(ctr * ctr, axis=1, keepdims=True)
    zt = ctr / jnp.sqrt(var + 1e-5) * gamma_ref[...] + beta_ref[...]
    zt_ref[...] = jnp.maximum(zt, 0.0)


def _tc_bn(acc, w_l, bias_col, gamma_col, beta_col):
    return pl.pallas_call(
        _tc_bn_body,
        in_specs=[
            pl.BlockSpec((3, NW, N_NODES), lambda: (0, 0, 0)),
            pl.BlockSpec((2, D), lambda: (0, 0)),
            pl.BlockSpec((D, 1), lambda: (0, 0)),
            pl.BlockSpec((D, 1), lambda: (0, 0)),
            pl.BlockSpec((D, 1), lambda: (0, 0)),
        ],
        out_specs=pl.BlockSpec((D, N_NODES), lambda: (0, 0)),
        out_shape=jax.ShapeDtypeStruct((D, N_NODES), jnp.float32),
    )(acc, w_l, bias_col, gamma_col, beta_col)


def _tc_decode_body(zt_ref, pi_ref):
    zg = zt_ref[...].reshape(D, GRAPH_SIZE).astype(jnp.bfloat16)
    lg = lax.dot_general(zg, zg, (((0,), (0,)), ((), ())),
                         preferred_element_type=jnp.float32)
    mx = jnp.max(lg, axis=0, keepdims=True)
    e = jnp.exp(lg - mx)
    s = jnp.sum(e, axis=0, keepdims=True)
    pi_ref[...] = (e / s)[None]


def _tc_decode(zt):
    return pl.pallas_call(
        _tc_decode_body,
        grid=(NG,),
        in_specs=[pl.BlockSpec((D, 1, 1, GRAPH_SIZE), lambda g: (0, g, 0, 0))],
        out_specs=pl.BlockSpec((1, GRAPH_SIZE, GRAPH_SIZE),
                               lambda g: (g, 0, 0)),
        out_shape=jax.ShapeDtypeStruct((NG, GRAPH_SIZE, GRAPH_SIZE),
                                       jnp.float32),
    )(zt)


def kernel(x, edge_index, edge_attributes, W_l, W_r, W_e, att, bias,
           bn_gamma, bn_beta):
    bf = lambda a: lax.optimization_barrier(
        a.astype(jnp.bfloat16)).astype(jnp.float32)
    xflat = bf(x).reshape(-1)
    src = edge_index[0].astype(jnp.int32)
    dst = edge_index[1].astype(jnp.int32)
    ea_rows = edge_attributes.astype(jnp.float32).reshape(NB, 1, BLK)
    wl_q = bf(W_l)

    v4t = jnp.concatenate([wl_q, bf(W_r)], axis=0).T
    we_col = W_e.astype(jnp.float32).reshape(1, D).T
    att_col = att.astype(jnp.float32).reshape(1, D).T

    u4 = _sc_gather(xflat, src, dst).reshape(4, N_EDGES)
    ex = _tc_alpha(u4, ea_rows, v4t, we_col, att_col)
    acc = _sc_scatter(xflat, src, dst,
                      ex.reshape(-1)).reshape(3, NW, N_NODES)
    zt = _tc_bn(acc, wl_q, bias.astype(jnp.float32).reshape(D, 1),
                bn_gamma.reshape(D, 1), bn_beta.reshape(D, 1))
    pi = _tc_decode(zt.reshape(D, NG, 1, GRAPH_SIZE))
    return pi


# speedup vs baseline: 27.0582x; 1.0799x over previous
import functools

import jax
import jax.numpy as jnp
from jax import lax
from jax.experimental import pallas as pl
from jax.experimental.pallas import tpu as pltpu
from jax.experimental.pallas import tpu_sc as plsc

N_NODES = 10000
N_EDGES = 320000
D = 128
GRAPH_SIZE = 1000
NG = N_NODES // GRAPH_SIZE

NW = 32
NE_PER = N_EDGES // NW
VREGS = NE_PER // 16

BLK = 2560
NB = N_EDGES // BLK

_mesh = plsc.VectorSubcoreMesh(core_axis_name="c", subcore_axis_name="s")
_sc_params = pltpu.CompilerParams(needs_layout_passes=False)


@functools.partial(
    pl.kernel,
    mesh=_mesh,
    out_type=jax.ShapeDtypeStruct((4 * N_EDGES,), jnp.float32),
    compiler_params=_sc_params,
    scratch_types=[
        pltpu.VMEM((2 * N_NODES,), jnp.float32),
        pltpu.VMEM((NE_PER,), jnp.int32),
        pltpu.VMEM((NE_PER,), jnp.int32),
        pltpu.VMEM((NE_PER,), jnp.float32),
        pltpu.VMEM((NE_PER,), jnp.float32),
        pltpu.VMEM((NE_PER,), jnp.float32),
        pltpu.VMEM((NE_PER,), jnp.float32),
    ],
)
def _sc_gather(xflat_hbm, src_hbm, dst_hbm, u4_hbm,
               xflat_v, src_v, dst_v, u0_v, u1_v, u2_v, u3_v):
    wid = lax.axis_index("s") * 2 + lax.axis_index("c")
    base = wid * NE_PER
    pltpu.sync_copy(xflat_hbm, xflat_v)
    pltpu.sync_copy(src_hbm.at[pl.ds(base, NE_PER)], src_v)
    pltpu.sync_copy(dst_hbm.at[pl.ds(base, NE_PER)], dst_v)

    def body(i, _):
        for j in range(5):
            sl = pl.ds((i * 5 + j) * 16, 16)
            s2 = src_v[sl] * 2
            d2 = dst_v[sl] * 2
            u0_v[sl] = plsc.load_gather(xflat_v, [s2])
            u1_v[sl] = plsc.load_gather(xflat_v, [s2 + 1])
            u2_v[sl] = plsc.load_gather(xflat_v, [d2])
            u3_v[sl] = plsc.load_gather(xflat_v, [d2 + 1])
        return 0

    lax.fori_loop(0, VREGS // 5, body, 0)
    pltpu.sync_copy(u0_v, u4_hbm.at[pl.ds(0 * N_EDGES + base, NE_PER)])
    pltpu.sync_copy(u1_v, u4_hbm.at[pl.ds(1 * N_EDGES + base, NE_PER)])
    pltpu.sync_copy(u2_v, u4_hbm.at[pl.ds(2 * N_EDGES + base, NE_PER)])
    pltpu.sync_copy(u3_v, u4_hbm.at[pl.ds(3 * N_EDGES + base, NE_PER)])


def _tc_alpha_body(u4_ref, ea_ref, v4t_ref, we_ref, att_ref, ex_ref):
    u = u4_ref[...]
    vt = v4t_ref[...]
    m = lax.dot_general(vt, u, (((1,), (0,)), ((), ())),
                        preferred_element_type=jnp.float32)
    m = m + we_ref[...] * ea_ref[0]
    m = jnp.where(m > 0, m, m * 0.2)
    t = m * att_ref[...]
    alpha = jnp.sum(t, axis=0, keepdims=True)
    ex_ref[0] = jnp.exp(alpha)


def _tc_alpha(u4, ea_rows, v4t, we_col, att_col):
    return pl.pallas_call(
        _tc_alpha_body,
        grid=(NB,),
        in_specs=[
            pl.BlockSpec((4, BLK), lambda i: (0, i)),
            pl.BlockSpec((1, 1, BLK), lambda i: (i, 0, 0)),
            pl.BlockSpec((D, 4), lambda i: (0, 0)),
            pl.BlockSpec((D, 1), lambda i: (0, 0)),
            pl.BlockSpec((D, 1), lambda i: (0, 0)),
        ],
        out_specs=pl.BlockSpec((1, 1, BLK), lambda i: (i, 0, 0)),
        out_shape=jax.ShapeDtypeStruct((NB, 1, BLK), jnp.float32),
    )(u4, ea_rows, v4t, we_col, att_col)


@functools.partial(
    pl.kernel,
    mesh=_mesh,
    out_type=jax.ShapeDtypeStruct((3 * NW * N_NODES,), jnp.float32),
    compiler_params=_sc_params,
    scratch_types=[
        pltpu.VMEM((2 * N_NODES,), jnp.float32),
        pltpu.VMEM((NE_PER,), jnp.int32),
        pltpu.VMEM((NE_PER,), jnp.int32),
        pltpu.VMEM((NE_PER,), jnp.float32),
        pltpu.VMEM((3 * N_NODES,), jnp.float32),
    ],
)
def _sc_scatter(xflat_hbm, src_hbm, dst_hbm, ex_hbm, acc_hbm,
                xflat_v, src_v, dst_v, ex_v, acc_v):
    wid = lax.axis_index("s") * 2 + lax.axis_index("c")
    base = wid * NE_PER
    pltpu.sync_copy(xflat_hbm, xflat_v)
    pltpu.sync_copy(src_hbm.at[pl.ds(base, NE_PER)], src_v)
    pltpu.sync_copy(dst_hbm.at[pl.ds(base, NE_PER)], dst_v)
    pltpu.sync_copy(ex_hbm.at[pl.ds(base, NE_PER)], ex_v)

    def zero(i, _):
        for j in range(15):
            acc_v[pl.ds((i * 15 + j) * 16, 16)] = jnp.zeros((16,), jnp.float32)
        return 0

    lax.fori_loop(0, (3 * N_NODES) // (16 * 15), zero, 0)

    def body(i, _):
        for j in range(5):
            sl = pl.ds((i * 5 + j) * 16, 16)
            s2 = src_v[sl] * 2
            d = dst_v[sl]
            ex = ex_v[sl]
            xs0 = plsc.load_gather(xflat_v, [s2])
            xs1 = plsc.load_gather(xflat_v, [s2 + 1])
            v1 = ex * xs0
            v2 = ex * xs1
            cnt, _ = plsc.scan_count(d)
            base = jnp.min(cnt)
            first = cnt == base
            plsc.addupdate_scatter(acc_v, [d], ex, mask=first)
            plsc.addupdate_scatter(acc_v, [d + N_NODES], v1, mask=first)
            plsc.addupdate_scatter(acc_v, [d + 2 * N_NODES], v2, mask=first)

            @pl.when(jnp.any(cnt != base))
            def _slow():
                def dup(k, _):
                    m = cnt == base + k
                    plsc.addupdate_scatter(acc_v, [d], ex, mask=m)
                    plsc.addupdate_scatter(acc_v, [d + N_NODES], v1, mask=m)
                    plsc.addupdate_scatter(acc_v, [d + 2 * N_NODES], v2,
                                           mask=m)
                    return 0
                lax.fori_loop(1, 16, dup, 0)
        return 0

    lax.fori_loop(0, VREGS // 5, body, 0)
    for c in range(3):
        pltpu.sync_copy(
            acc_v.at[pl.ds(c * N_NODES, N_NODES)],
            acc_hbm.at[pl.ds((c * NW + wid) * N_NODES, N_NODES)])


def _tc_decode_body(acc_ref, wlt_ref, bias_ref, gamma_ref, beta_ref,
                    pi_ref, zt_s):
    g = pl.program_id(0)

    @pl.when(g == 0)
    def _init():
        a = acc_ref[...]
        r = jnp.sum(a, axis=1)
        inv = 1.0 / (r[0] + 1e-16)
        sx = (r[1] * inv)[:, None, :]
        sy = (r[2] * inv)[:, None, :]
        w0 = wlt_ref[:, 0:1][None]
        w1 = wlt_ref[:, 1:2][None]
        out = sx * w0 + sy * w1 + bias_ref[...][None]
        mu = jnp.sum(jnp.sum(out, axis=0, keepdims=True),
                     axis=2, keepdims=True) / N_NODES
        ctr = out - mu
        var = jnp.sum(jnp.sum(ctr * ctr, axis=0, keepdims=True),
                      axis=2, keepdims=True) / N_NODES
        zt = ctr / jnp.sqrt(var + 1e-5) * gamma_ref[...][None] \
            + beta_ref[...][None]
        zt_s[...] = jnp.maximum(zt, 0.0)

    zg = zt_s[g].astype(jnp.bfloat16)
    lg = lax.dot_general(zg, zg, (((0,), (0,)), ((), ())),
                         preferred_element_type=jnp.float32)
    mx = jnp.max(lg, axis=0, keepdims=True)
    e = jnp.exp(lg - mx)
    s = jnp.sum(e, axis=0, keepdims=True)
    pi_ref[...] = (e / s)[None]


def _tc_decode(acc4, wlt, bias_col, gamma_col, beta_col):
    return pl.pallas_call(
        _tc_decode_body,
        grid=(NG,),
        in_specs=[
            pl.BlockSpec((3, NW, NG, GRAPH_SIZE), lambda g: (0, 0, 0, 0)),
            pl.BlockSpec((D, 2), lambda g: (0, 0)),
            pl.BlockSpec((D, 1), lambda g: (0, 0)),
            pl.BlockSpec((D, 1), lambda g: (0, 0)),
            pl.BlockSpec((D, 1), lambda g: (0, 0)),
        ],
        out_specs=pl.BlockSpec((1, GRAPH_SIZE, GRAPH_SIZE),
                               lambda g: (g, 0, 0)),
        out_shape=jax.ShapeDtypeStruct((NG, GRAPH_SIZE, GRAPH_SIZE),
                                       jnp.float32),
        scratch_shapes=[pltpu.VMEM((NG, D, GRAPH_SIZE), jnp.float32)],
    )(acc4, wlt, bias_col, gamma_col, beta_col)


def kernel(x, edge_index, edge_attributes, W_l, W_r, W_e, att, bias,
           bn_gamma, bn_beta):
    bf = lambda a: lax.optimization_barrier(
        a.astype(jnp.bfloat16)).astype(jnp.float32)
    xflat = bf(x).reshape(-1)
    src = edge_index[0].astype(jnp.int32)
    dst = edge_index[1].astype(jnp.int32)
    ea_rows = edge_attributes.astype(jnp.float32).reshape(NB, 1, BLK)
    wl_q = bf(W_l)

    v4t = jnp.concatenate([wl_q, bf(W_r)], axis=0).T
    we_col = W_e.astype(jnp.float32).reshape(1, D).T
    att_col = att.astype(jnp.float32).reshape(1, D).T

    u4 = _sc_gather(xflat, src, dst).reshape(4, N_EDGES)
    ex = _tc_alpha(u4, ea_rows, v4t, we_col, att_col)
    acc4 = _sc_scatter(xflat, src, dst,
                       ex.reshape(-1)).reshape(3, NW, NG, GRAPH_SIZE)
    pi = _tc_decode(acc4, wl_q.T,
                    bias.astype(jnp.float32).reshape(D, 1),
                    bn_gamma.reshape(D, 1), bn_beta.reshape(D, 1))
    return pi


# speedup vs baseline: 28.5892x; 1.0566x over previous
import functools

import jax
import jax.numpy as jnp
from jax import lax
from jax.experimental import pallas as pl
from jax.experimental.pallas import tpu as pltpu
from jax.experimental.pallas import tpu_sc as plsc

N_NODES = 10000
N_EDGES = 320000
D = 128
GRAPH_SIZE = 1000
NG = N_NODES // GRAPH_SIZE

NW = 32
NE_PER = N_EDGES // NW
VREGS = NE_PER // 16

BLK = 2560
NB = N_EDGES // BLK

_mesh = plsc.VectorSubcoreMesh(core_axis_name="c", subcore_axis_name="s")
_sc_params = pltpu.CompilerParams(needs_layout_passes=False)


@functools.partial(
    pl.kernel,
    mesh=_mesh,
    out_type=jax.ShapeDtypeStruct((4 * N_EDGES,), jnp.float32),
    compiler_params=_sc_params,
    scratch_types=[
        pltpu.VMEM((2 * N_NODES,), jnp.float32),
        pltpu.VMEM((NE_PER,), jnp.int32),
        pltpu.VMEM((NE_PER,), jnp.int32),
        pltpu.VMEM((NE_PER,), jnp.float32),
        pltpu.VMEM((NE_PER,), jnp.float32),
        pltpu.VMEM((NE_PER,), jnp.float32),
        pltpu.VMEM((NE_PER,), jnp.float32),
    ],
)
def _sc_gather(xflat_hbm, src_hbm, dst_hbm, u4_hbm,
               xflat_v, src_v, dst_v, u0_v, u1_v, u2_v, u3_v):
    wid = lax.axis_index("s") * 2 + lax.axis_index("c")
    base = wid * NE_PER
    pltpu.sync_copy(xflat_hbm, xflat_v)
    pltpu.sync_copy(src_hbm.at[pl.ds(base, NE_PER)], src_v)
    pltpu.sync_copy(dst_hbm.at[pl.ds(base, NE_PER)], dst_v)

    def body(i, _):
        for j in range(5):
            sl = pl.ds((i * 5 + j) * 16, 16)
            s2 = src_v[sl] * 2
            d2 = dst_v[sl] * 2
            u0_v[sl] = plsc.load_gather(xflat_v, [s2])
            u1_v[sl] = plsc.load_gather(xflat_v, [s2 + 1])
            u2_v[sl] = plsc.load_gather(xflat_v, [d2])
            u3_v[sl] = plsc.load_gather(xflat_v, [d2 + 1])
        return 0

    lax.fori_loop(0, VREGS // 5, body, 0)
    pltpu.sync_copy(u0_v, u4_hbm.at[pl.ds(0 * N_EDGES + base, NE_PER)])
    pltpu.sync_copy(u1_v, u4_hbm.at[pl.ds(1 * N_EDGES + base, NE_PER)])
    pltpu.sync_copy(u2_v, u4_hbm.at[pl.ds(2 * N_EDGES + base, NE_PER)])
    pltpu.sync_copy(u3_v, u4_hbm.at[pl.ds(3 * N_EDGES + base, NE_PER)])


def _tc_alpha_body(u4_ref, ea_ref, v4t_ref, we_ref, att_ref, ex_ref):
    u = u4_ref[...]
    vt = v4t_ref[...]
    m = lax.dot_general(vt, u, (((1,), (0,)), ((), ())),
                        preferred_element_type=jnp.float32)
    m = m + we_ref[...] * ea_ref[0]
    m = jnp.where(m > 0, m, m * 0.2)
    t = m * att_ref[...]
    alpha = jnp.sum(t, axis=0, keepdims=True)
    ex_ref[0] = jnp.exp(alpha)


def _tc_alpha(u4, ea_rows, v4t, we_col, att_col):
    return pl.pallas_call(
        _tc_alpha_body,
        grid=(NB,),
        in_specs=[
            pl.BlockSpec((4, BLK), lambda i: (0, i)),
            pl.BlockSpec((1, 1, BLK), lambda i: (i, 0, 0)),
            pl.BlockSpec((D, 4), lambda i: (0, 0)),
            pl.BlockSpec((D, 1), lambda i: (0, 0)),
            pl.BlockSpec((D, 1), lambda i: (0, 0)),
        ],
        out_specs=pl.BlockSpec((1, 1, BLK), lambda i: (i, 0, 0)),
        out_shape=jax.ShapeDtypeStruct((NB, 1, BLK), jnp.float32),
    )(u4, ea_rows, v4t, we_col, att_col)


@functools.partial(
    pl.kernel,
    mesh=_mesh,
    out_type=jax.ShapeDtypeStruct((3 * NW * N_NODES,), jnp.float32),
    compiler_params=_sc_params,
    scratch_types=[
        pltpu.VMEM((NE_PER,), jnp.float32),
        pltpu.VMEM((NE_PER,), jnp.float32),
        pltpu.VMEM((NE_PER,), jnp.int32),
        pltpu.VMEM((NE_PER,), jnp.float32),
        pltpu.VMEM((3 * N_NODES,), jnp.float32),
    ],
)
def _sc_scatter(u4_hbm, dst_hbm, ex_hbm, acc_hbm,
                xs0_v, xs1_v, dst_v, ex_v, acc_v):
    wid = lax.axis_index("s") * 2 + lax.axis_index("c")
    base = wid * NE_PER
    pltpu.sync_copy(u4_hbm.at[pl.ds(base, NE_PER)], xs0_v)
    pltpu.sync_copy(u4_hbm.at[pl.ds(N_EDGES + base, NE_PER)], xs1_v)
    pltpu.sync_copy(dst_hbm.at[pl.ds(base, NE_PER)], dst_v)
    pltpu.sync_copy(ex_hbm.at[pl.ds(base, NE_PER)], ex_v)

    def zero(i, _):
        for j in range(15):
            acc_v[pl.ds((i * 15 + j) * 16, 16)] = jnp.zeros((16,), jnp.float32)
        return 0

    lax.fori_loop(0, (3 * N_NODES) // (16 * 15), zero, 0)

    def body(i, _):
        for j in range(5):
            sl = pl.ds((i * 5 + j) * 16, 16)
            d = dst_v[sl]
            ex = ex_v[sl]
            v1 = ex * xs0_v[sl]
            v2 = ex * xs1_v[sl]
            cnt, _ = plsc.scan_count(d)
            first = cnt == 1
            plsc.addupdate_scatter(acc_v, [d], ex, mask=first)
            plsc.addupdate_scatter(acc_v, [d + N_NODES], v1, mask=first)
            plsc.addupdate_scatter(acc_v, [d + 2 * N_NODES], v2, mask=first)

            @pl.when(jnp.any(cnt != 1))
            def _slow():
                def dup(k, _):
                    m = cnt == 1 + k
                    plsc.addupdate_scatter(acc_v, [d], ex, mask=m)
                    plsc.addupdate_scatter(acc_v, [d + N_NODES], v1, mask=m)
                    plsc.addupdate_scatter(acc_v, [d + 2 * N_NODES], v2,
                                           mask=m)
                    return 0
                lax.fori_loop(1, 16, dup, 0)
        return 0

    lax.fori_loop(0, VREGS // 5, body, 0)
    for c in range(3):
        pltpu.sync_copy(
            acc_v.at[pl.ds(c * N_NODES, N_NODES)],
            acc_hbm.at[pl.ds((c * NW + wid) * N_NODES, N_NODES)])


def _tc_decode_body(acc_ref, wlt_ref, bias_ref, gamma_ref, beta_ref,
                    pi_ref, zt_s):
    g = pl.program_id(0)

    @pl.when(g == 0)
    def _init():
        a = acc_ref[...]
        r = jnp.sum(a, axis=1)
        inv = 1.0 / (r[0] + 1e-16)
        sx = (r[1] * inv)[:, None, :]
        sy = (r[2] * inv)[:, None, :]
        w0 = wlt_ref[:, 0:1][None]
        w1 = wlt_ref[:, 1:2][None]
        out = sx * w0 + sy * w1 + bias_ref[...][None]
        mu = jnp.sum(jnp.sum(out, axis=0, keepdims=True),
                     axis=2, keepdims=True) / N_NODES
        ctr = out - mu
        var = jnp.sum(jnp.sum(ctr * ctr, axis=0, keepdims=True),
                      axis=2, keepdims=True) / N_NODES
        zt = ctr / jnp.sqrt(var + 1e-5) * gamma_ref[...][None] \
            + beta_ref[...][None]
        zt_s[...] = jnp.maximum(zt, 0.0)

    zg = zt_s[g].astype(jnp.bfloat16)
    lg = lax.dot_general(zg, zg, (((0,), (0,)), ((), ())),
                         preferred_element_type=jnp.float32)
    mx = jnp.max(lg, axis=0, keepdims=True)
    e = jnp.exp(lg - mx)
    s = jnp.sum(e, axis=0, keepdims=True)
    pi_ref[...] = (e / s)[None]


def _tc_decode(acc4, wlt, bias_col, gamma_col, beta_col):
    return pl.pallas_call(
        _tc_decode_body,
        grid=(NG,),
        in_specs=[
            pl.BlockSpec((3, NW, NG, GRAPH_SIZE), lambda g: (0, 0, 0, 0)),
            pl.BlockSpec((D, 2), lambda g: (0, 0)),
            pl.BlockSpec((D, 1), lambda g: (0, 0)),
            pl.BlockSpec((D, 1), lambda g: (0, 0)),
            pl.BlockSpec((D, 1), lambda g: (0, 0)),
        ],
        out_specs=pl.BlockSpec((1, GRAPH_SIZE, GRAPH_SIZE),
                               lambda g: (g, 0, 0)),
        out_shape=jax.ShapeDtypeStruct((NG, GRAPH_SIZE, GRAPH_SIZE),
                                       jnp.float32),
        scratch_shapes=[pltpu.VMEM((NG, D, GRAPH_SIZE), jnp.float32)],
    )(acc4, wlt, bias_col, gamma_col, beta_col)


def kernel(x, edge_index, edge_attributes, W_l, W_r, W_e, att, bias,
           bn_gamma, bn_beta):
    bf = lambda a: lax.optimization_barrier(
        a.astype(jnp.bfloat16)).astype(jnp.float32)
    xflat = bf(x).reshape(-1)
    src = edge_index[0].astype(jnp.int32)
    dst = edge_index[1].astype(jnp.int32)
    ea_rows = edge_attributes.astype(jnp.float32).reshape(NB, 1, BLK)
    wl_q = bf(W_l)

    v4t = jnp.concatenate([wl_q, bf(W_r)], axis=0).T
    we_col = W_e.astype(jnp.float32).reshape(1, D).T
    att_col = att.astype(jnp.float32).reshape(1, D).T

    u4flat = _sc_gather(xflat, src, dst)
    ex = _tc_alpha(u4flat.reshape(4, N_EDGES), ea_rows,
                   v4t, we_col, att_col)
    acc4 = _sc_scatter(u4flat, dst,
                       ex.reshape(-1)).reshape(3, NW, NG, GRAPH_SIZE)
    pi = _tc_decode(acc4, wl_q.T,
                    bias.astype(jnp.float32).reshape(D, 1),
                    bn_gamma.reshape(D, 1), bn_beta.reshape(D, 1))
    return pi
